# baseline (speedup 1.0000x reference)
import functools

import jax
import jax.numpy as jnp
from jax import lax
from jax.experimental import pallas as pl
from jax.experimental.pallas import tpu as pltpu
from jax.experimental.pallas import tpu_sc as plsc

N = 10000
E = 160000
H = 256
HH = H // 2
G = 64

NP = 10240
NC, NS = 2, 16
STRIPE = NP // NS

EPT2 = E // NS
BLK2 = 80
NBLK2 = EPT2 // BLK2


def _agg_body(h_hbm, src_hbm, dst_hbm, out_hbm, src_v, dst_v, buf0,
              buf1, g0, g1, acc):
    c = lax.axis_index("c")
    s = lax.axis_index("s")

    pltpu.sync_copy(src_hbm.at[s], src_v)
    pltpu.sync_copy(dst_hbm.at[s], dst_v)

    z = jnp.zeros((16,), jnp.float32)

    def zero_row(r, _):
        for k in range(HH // 16):
            buf0[r, pl.ds(k * 16, 16)] = z
        return 0

    lax.fori_loop(0, BLK2, zero_row, 0)
    for q in range(STRIPE // BLK2):
        pltpu.sync_copy(buf0, acc.at[pl.ds(s * STRIPE + q * BLK2, BLK2)])

    plsc.subcore_barrier()

    def gath(b, buf, sem):
        pltpu.async_copy(
            h_hbm.at[c].at[src_v.at[pl.ds(b * BLK2, BLK2)]], buf, sem)

    def wait_g(buf, sem):
        pltpu.make_async_copy(h_hbm.at[c], buf, sem).wait()

    def scat(b, buf):
        pltpu.sync_copy(buf, acc.at[dst_v.at[b]], add=True)

    gath(0, buf0, g0)

    def pair(t, _):
        e = 2 * t
        gath(e + 1, buf1, g1)
        wait_g(buf0, g0)
        scat(e, buf0)
        gath(e + 2, buf0, g0)
        wait_g(buf1, g1)
        scat(e + 1, buf1)
        return 0

    lax.fori_loop(0, (NBLK2 - 1) // 2, pair, 0)
    wait_g(buf0, g0)
    scat(NBLK2 - 1, buf0)

    plsc.subcore_barrier()

    pltpu.sync_copy(acc.at[pl.ds(s * STRIPE, STRIPE)],
                    out_hbm.at[c, pl.ds(s * STRIPE, STRIPE)])


@functools.lru_cache(maxsize=None)
def _get_agg():
    return pl.kernel(
        _agg_body,
        out_type=jax.ShapeDtypeStruct((NC, NP, HH), jnp.float32),
        mesh=plsc.VectorSubcoreMesh(core_axis_name="c", subcore_axis_name="s",
                                    num_cores=NC, num_subcores=NS),
        scratch_types=[
            pltpu.VMEM((EPT2,), jnp.int32),
            pltpu.VMEM((NBLK2, BLK2), jnp.int32),
            pltpu.VMEM((BLK2, HH), jnp.float32),
            pltpu.VMEM((BLK2, HH), jnp.float32),
            pltpu.SemaphoreType.DMA,
            pltpu.SemaphoreType.DMA,
            pltpu.VMEM_SHARED((NP, HH), jnp.float32),
        ],
    )


EPT0 = E // (NC * NS)
BLK0 = 100
NBLK0 = EPT0 // BLK0


def _agg0_body(x_hbm, src_hbm, dst_hbm, out_hbm, src_v, dst_v, buf0, buf1,
               g0, g1, acc):
    c = lax.axis_index("c")
    s = lax.axis_index("s")
    w = c * NS + s

    pltpu.sync_copy(src_hbm.at[w], src_v)
    pltpu.sync_copy(dst_hbm.at[w], dst_v)

    z = jnp.zeros((16,), jnp.float32)

    def zero_row(r, _):
        for k in range(HH // 16):
            buf0[r, pl.ds(k * 16, 16)] = z
        return 0

    lax.fori_loop(0, BLK0, zero_row, 0)
    for q in range(STRIPE // BLK0):
        pltpu.sync_copy(buf0, acc.at[pl.ds(s * STRIPE + q * BLK0, BLK0)])
    rem = STRIPE % BLK0
    if rem:
        pltpu.sync_copy(
            buf0.at[pl.ds(0, rem)],
            acc.at[pl.ds(s * STRIPE + (STRIPE // BLK0) * BLK0, rem)])

    plsc.subcore_barrier()

    def gath(b, buf, sem):
        return pltpu.async_copy(x_hbm.at[src_v.at[b]], buf, sem)

    def scat(b, buf):
        pltpu.sync_copy(buf, acc.at[dst_v.at[b]], add=True)

    gath(0, buf0, g0)

    def pair(t, _):
        e = 2 * t
        gath(e + 1, buf1, g1)
        pltpu.make_async_copy(x_hbm, buf0, g0).wait()
        scat(e, buf0)
        gath(e + 2, buf0, g0)
        pltpu.make_async_copy(x_hbm, buf1, g1).wait()
        scat(e + 1, buf1)
        return 0

    lax.fori_loop(0, (NBLK0 - 2) // 2, pair, 0)
    gath(NBLK0 - 1, buf1, g1)
    pltpu.make_async_copy(x_hbm, buf0, g0).wait()
    scat(NBLK0 - 2, buf0)
    pltpu.make_async_copy(x_hbm, buf1, g1).wait()
    scat(NBLK0 - 1, buf1)

    plsc.subcore_barrier()

    pltpu.sync_copy(acc.at[pl.ds(s * STRIPE, STRIPE)],
                    out_hbm.at[c, pl.ds(s * STRIPE, STRIPE)])


@functools.lru_cache(maxsize=None)
def _get_agg0():
    return pl.kernel(
        _agg0_body,
        out_type=jax.ShapeDtypeStruct((NC, NP, HH), jnp.float32),
        mesh=plsc.VectorSubcoreMesh(core_axis_name="c", subcore_axis_name="s",
                                    num_cores=NC, num_subcores=NS),
        scratch_types=[
            pltpu.VMEM((NBLK0, BLK0), jnp.int32),
            pltpu.VMEM((NBLK0, BLK0), jnp.int32),
            pltpu.VMEM((BLK0, HH), jnp.float32),
            pltpu.VMEM((BLK0, HH), jnp.float32),
            pltpu.SemaphoreType.DMA,
            pltpu.SemaphoreType.DMA,
            pltpu.VMEM_SHARED((NP, HH), jnp.float32),
        ],
    )


def _agg(h, src, dst):
    return _get_agg()(h, src, dst)


def _agg0(x0, src, dst):
    return _get_agg0()(x0, src, dst)


BN = 512
NBN = NP // BN


def _dotbf(a, b):
    return jnp.dot(a.astype(jnp.bfloat16), b.astype(jnp.bfloat16),
                   preferred_element_type=jnp.float32)


def _pre0_body(x_ref, w_ref, b_ref, o_ref):
    o = _dotbf(x_ref[...], w_ref[...]) + b_ref[...]
    o_ref[0] = o[:, :HH]
    o_ref[1] = o[:, HH:]


_pre0 = pl.pallas_call(
    _pre0_body,
    grid=(NBN,),
    in_specs=[
        pl.BlockSpec((BN, HH), lambda i: (i, 0)),
        pl.BlockSpec((HH, H), lambda i: (0, 0)),
        pl.BlockSpec((1, H), lambda i: (0, 0)),
    ],
    out_specs=pl.BlockSpec((NC, BN, HH), lambda i: (0, i, 0)),
    out_shape=jax.ShapeDtypeStruct((NC, NP, HH), jnp.float32),
)


def _fused0_body(s_ref, t_ref, wl_ref, wr_ref, b_ref, h_ref, tn_ref):
    sfull = s_ref[0] + s_ref[1]
    deg = sfull[:, 2:3]
    inv = 1.0 / jnp.maximum(deg, 1.0)
    tfull = jnp.concatenate([t_ref[0], t_ref[1]], axis=1)
    h = _dotbf(sfull * inv, wl_ref[...]) + tfull
    h = jnp.maximum(h, 0.0)
    h_ref[0] = h[:, :HH]
    h_ref[1] = h[:, HH:]
    tn = _dotbf(h, wr_ref[...]) + b_ref[...]
    tn_ref[0] = tn[:, :HH]
    tn_ref[1] = tn[:, HH:]


_fused0 = pl.pallas_call(
    _fused0_body,
    grid=(NBN,),
    in_specs=[
        pl.BlockSpec((NC, BN, HH), lambda i: (0, i, 0)),
        pl.BlockSpec((NC, BN, HH), lambda i: (0, i, 0)),
        pl.BlockSpec((HH, H), lambda i: (0, 0)),
        pl.BlockSpec((H, H), lambda i: (0, 0)),
        pl.BlockSpec((1, H), lambda i: (0, 0)),
    ],
    out_specs=[
        pl.BlockSpec((NC, BN, HH), lambda i: (0, i, 0)),
        pl.BlockSpec((NC, BN, HH), lambda i: (0, i, 0)),
    ],
    out_shape=[
        jax.ShapeDtypeStruct((NC, NP, HH), jnp.float32),
        jax.ShapeDtypeStruct((NC, NP, HH), jnp.float32),
    ],
)


def _fused_body(s_ref, t_ref, s0_ref, wl_ref, wr_ref, b_ref, h_ref, tn_ref):
    sfull = jnp.concatenate([s_ref[0], s_ref[1]], axis=1)
    deg = s0_ref[0][:, 2:3] + s0_ref[1][:, 2:3]
    inv = 1.0 / jnp.maximum(deg, 1.0)
    tfull = jnp.concatenate([t_ref[0], t_ref[1]], axis=1)
    h = _dotbf(sfull * inv, wl_ref[...]) + tfull
    h = jnp.maximum(h, 0.0)
    h_ref[0] = h[:, :HH]
    h_ref[1] = h[:, HH:]
    tn = _dotbf(h, wr_ref[...]) + b_ref[...]
    tn_ref[0] = tn[:, :HH]
    tn_ref[1] = tn[:, HH:]


_fused = pl.pallas_call(
    _fused_body,
    grid=(NBN,),
    in_specs=[
        pl.BlockSpec((NC, BN, HH), lambda i: (0, i, 0)),
        pl.BlockSpec((NC, BN, HH), lambda i: (0, i, 0)),
        pl.BlockSpec((NC, BN, HH), lambda i: (0, i, 0)),
        pl.BlockSpec((H, H), lambda i: (0, 0)),
        pl.BlockSpec((H, H), lambda i: (0, 0)),
        pl.BlockSpec((1, H), lambda i: (0, 0)),
    ],
    out_specs=[
        pl.BlockSpec((NC, BN, HH), lambda i: (0, i, 0)),
        pl.BlockSpec((NC, BN, HH), lambda i: (0, i, 0)),
    ],
    out_shape=[
        jax.ShapeDtypeStruct((NC, NP, HH), jnp.float32),
        jax.ShapeDtypeStruct((NC, NP, HH), jnp.float32),
    ],
)


def _final_body(s_ref, t_ref, s0_ref, bt_ref, wl_ref, o_ref, acc, cnt):
    i = pl.program_id(0)

    @pl.when(i == 0)
    def _():
        acc[...] = jnp.zeros_like(acc)
        cnt[...] = jnp.zeros_like(cnt)

    sfull = jnp.concatenate([s_ref[0], s_ref[1]], axis=1)
    deg = s0_ref[0][:, 2:3] + s0_ref[1][:, 2:3]
    inv = 1.0 / jnp.maximum(deg, 1.0)
    tfull = jnp.concatenate([t_ref[0], t_ref[1]], axis=1)
    h2 = _dotbf(sfull * inv, wl_ref[...]) + tfull
    bt = bt_ref[0, 0, :]
    p = (lax.broadcasted_iota(jnp.int32, (G, BN), 0) == bt[None, :]).astype(
        jnp.float32)
    acc[...] += jnp.dot(p, h2, preferred_element_type=jnp.float32)
    cnt[...] += jnp.sum(p, axis=1, keepdims=True)

    @pl.when(i == NBN - 1)
    def _():
        o_ref[...] = acc[...] / jnp.maximum(cnt[...], 1.0)


_final = pl.pallas_call(
    _final_body,
    grid=(NBN,),
    in_specs=[
        pl.BlockSpec((NC, BN, HH), lambda i: (0, i, 0)),
        pl.BlockSpec((NC, BN, HH), lambda i: (0, i, 0)),
        pl.BlockSpec((NC, BN, HH), lambda i: (0, i, 0)),
        pl.BlockSpec((1, 1, BN), lambda i: (i, 0, 0)),
        pl.BlockSpec((H, H), lambda i: (0, 0)),
    ],
    out_specs=pl.BlockSpec((G, H), lambda i: (0, 0)),
    out_shape=jax.ShapeDtypeStruct((G, H), jnp.float32),
    scratch_shapes=[
        pltpu.VMEM((G, H), jnp.float32),
        pltpu.VMEM((G, 1), jnp.float32),
    ],
)


def kernel(x, edge_index, batch, W_l0, b0, W_r0, W_l1, b1, W_r1, W_l2, b2,
           W_r2):
    src = edge_index[0].reshape(NS, EPT2)
    dst = edge_index[1].reshape(NS, NBLK2, BLK2)
    src0 = edge_index[0].reshape(NC * NS, NBLK0, BLK0)
    dst0 = edge_index[1].reshape(NC * NS, NBLK0, BLK0)

    x0 = jnp.concatenate(
        [x, jnp.ones((N, 1), jnp.float32), jnp.zeros((N, HH - 3), jnp.float32)],
        axis=1)
    x0 = jnp.pad(x0, ((0, NP - N), (0, 0)))
    wl0 = jnp.pad(W_l0, ((0, HH - 2), (0, 0)))
    wr0 = jnp.pad(W_r0, ((0, HH - 2), (0, 0)))
    batch3 = jnp.pad(batch, (0, NP - N), constant_values=G).reshape(NBN, 1, BN)
    b0r, b1r, b2r = b0.reshape(1, H), b1.reshape(1, H), b2.reshape(1, H)

    s0 = _agg0(x0, src0, dst0)
    t0 = _pre0(x0, wr0, b0r)
    h0, t1 = _fused0(s0, t0, wl0, W_r1, b1r)
    s1 = _agg(h0, src, dst)
    h1, t2 = _fused(s1, t1, s0, W_l1, W_r2, b2r)
    s2 = _agg(h1, src, dst)
    return _final(s2, t2, s0, batch3, W_l2)


# speedup vs baseline: 1.0226x; 1.0226x over previous
import functools

import jax
import jax.numpy as jnp
from jax import lax
from jax.experimental import pallas as pl
from jax.experimental.pallas import tpu as pltpu
from jax.experimental.pallas import tpu_sc as plsc

N = 10000
E = 160000
H = 256
HH = H // 2
G = 64

NP = 10240
NC, NS = 2, 16
STRIPE = NP // NS

EPT2 = E // NS
BLK2 = 80
NBLK2 = EPT2 // BLK2


def _agg_body(h_hbm, src_hbm, dst_hbm, out_hbm, src_v, dst_v, buf0,
              buf1, g0, g1, acc):
    c = lax.axis_index("c")
    s = lax.axis_index("s")

    pltpu.sync_copy(src_hbm.at[s], src_v)
    pltpu.sync_copy(dst_hbm.at[s], dst_v)

    z = jnp.zeros((16,), jnp.float32)

    def zero_row(r, _):
        for k in range(HH // 16):
            buf0[r, pl.ds(k * 16, 16)] = z
        return 0

    lax.fori_loop(0, BLK2, zero_row, 0)
    for q in range(STRIPE // BLK2):
        pltpu.sync_copy(buf0, acc.at[pl.ds(s * STRIPE + q * BLK2, BLK2)])

    plsc.subcore_barrier()

    def gath(b, buf, sem):
        pltpu.async_copy(
            h_hbm.at[c].at[src_v.at[pl.ds(b * BLK2, BLK2)]], buf, sem)

    def wait_g(buf, sem):
        pltpu.make_async_copy(h_hbm.at[c], buf, sem).wait()

    def scat(b, buf):
        pltpu.sync_copy(buf, acc.at[dst_v.at[b]], add=True)

    gath(0, buf0, g0)

    def pair(t, _):
        e = 2 * t
        gath(e + 1, buf1, g1)
        wait_g(buf0, g0)
        scat(e, buf0)
        gath(e + 2, buf0, g0)
        wait_g(buf1, g1)
        scat(e + 1, buf1)
        return 0

    lax.fori_loop(0, (NBLK2 - 1) // 2, pair, 0)
    wait_g(buf0, g0)
    scat(NBLK2 - 1, buf0)

    plsc.subcore_barrier()

    pltpu.sync_copy(acc.at[pl.ds(s * STRIPE, STRIPE)],
                    out_hbm.at[c, pl.ds(s * STRIPE, STRIPE)])


@functools.lru_cache(maxsize=None)
def _get_agg():
    return pl.kernel(
        _agg_body,
        out_type=jax.ShapeDtypeStruct((NC, NP, HH), jnp.float32),
        mesh=plsc.VectorSubcoreMesh(core_axis_name="c", subcore_axis_name="s",
                                    num_cores=NC, num_subcores=NS),
        scratch_types=[
            pltpu.VMEM((EPT2,), jnp.int32),
            pltpu.VMEM((NBLK2, BLK2), jnp.int32),
            pltpu.VMEM((BLK2, HH), jnp.float32),
            pltpu.VMEM((BLK2, HH), jnp.float32),
            pltpu.SemaphoreType.DMA,
            pltpu.SemaphoreType.DMA,
            pltpu.VMEM_SHARED((NP, HH), jnp.float32),
        ],
    )


EPT0 = E // (NC * NS)
BLK0 = 100
NBLK0 = EPT0 // BLK0


def _agg0_body(x_hbm, src_hbm, dst_hbm, out_hbm, src_v, dst_v, buf0, buf1,
               g0, g1, acc):
    c = lax.axis_index("c")
    s = lax.axis_index("s")
    w = c * NS + s

    pltpu.sync_copy(src_hbm.at[w], src_v)
    pltpu.sync_copy(dst_hbm.at[w], dst_v)

    z = jnp.zeros((16,), jnp.float32)

    def zero_row(r, _):
        for k in range(HH // 16):
            buf0[r, pl.ds(k * 16, 16)] = z
        return 0

    lax.fori_loop(0, BLK0, zero_row, 0)
    for q in range(STRIPE // BLK0):
        pltpu.sync_copy(buf0, acc.at[pl.ds(s * STRIPE + q * BLK0, BLK0)])
    rem = STRIPE % BLK0
    if rem:
        pltpu.sync_copy(
            buf0.at[pl.ds(0, rem)],
            acc.at[pl.ds(s * STRIPE + (STRIPE // BLK0) * BLK0, rem)])

    plsc.subcore_barrier()

    def gath(b, buf, sem):
        return pltpu.async_copy(x_hbm.at[src_v.at[b]], buf, sem)

    def scat(b, buf):
        pltpu.sync_copy(buf, acc.at[dst_v.at[b]], add=True)

    gath(0, buf0, g0)

    def pair(t, _):
        e = 2 * t
        gath(e + 1, buf1, g1)
        pltpu.make_async_copy(x_hbm, buf0, g0).wait()
        scat(e, buf0)
        gath(e + 2, buf0, g0)
        pltpu.make_async_copy(x_hbm, buf1, g1).wait()
        scat(e + 1, buf1)
        return 0

    lax.fori_loop(0, (NBLK0 - 2) // 2, pair, 0)
    gath(NBLK0 - 1, buf1, g1)
    pltpu.make_async_copy(x_hbm, buf0, g0).wait()
    scat(NBLK0 - 2, buf0)
    pltpu.make_async_copy(x_hbm, buf1, g1).wait()
    scat(NBLK0 - 1, buf1)

    plsc.subcore_barrier()

    pltpu.sync_copy(acc.at[pl.ds(s * STRIPE, STRIPE)],
                    out_hbm.at[c, pl.ds(s * STRIPE, STRIPE)])


@functools.lru_cache(maxsize=None)
def _get_agg0():
    return pl.kernel(
        _agg0_body,
        out_type=jax.ShapeDtypeStruct((NC, NP, HH), jnp.float32),
        mesh=plsc.VectorSubcoreMesh(core_axis_name="c", subcore_axis_name="s",
                                    num_cores=NC, num_subcores=NS),
        scratch_types=[
            pltpu.VMEM((NBLK0, BLK0), jnp.int32),
            pltpu.VMEM((NBLK0, BLK0), jnp.int32),
            pltpu.VMEM((BLK0, HH), jnp.float32),
            pltpu.VMEM((BLK0, HH), jnp.float32),
            pltpu.SemaphoreType.DMA,
            pltpu.SemaphoreType.DMA,
            pltpu.VMEM_SHARED((NP, HH), jnp.float32),
        ],
    )


def _agg(h, src, dst):
    return _get_agg()(h, src, dst)


def _agg0(x0, src, dst):
    return _get_agg0()(x0, src, dst)


BN = 512
NBN = NP // BN


def _layer0_body(s_ref, x_ref, wl_ref, wr_ref, b_ref, o_ref):
    sfull = s_ref[0] + s_ref[1]
    deg = sfull[:, 2:3]
    inv = 1.0 / jnp.maximum(deg, 1.0)
    o = (jnp.dot(sfull * inv, wl_ref[...], preferred_element_type=jnp.float32)
         + jnp.dot(x_ref[...], wr_ref[...], preferred_element_type=jnp.float32)
         + b_ref[...])
    o = jnp.maximum(o, 0.0)
    o_ref[0] = o[:, :HH]
    o_ref[1] = o[:, HH:]


_layer0 = pl.pallas_call(
    _layer0_body,
    grid=(NBN,),
    in_specs=[
        pl.BlockSpec((NC, BN, HH), lambda i: (0, i, 0)),
        pl.BlockSpec((BN, HH), lambda i: (i, 0)),
        pl.BlockSpec((HH, H), lambda i: (0, 0)),
        pl.BlockSpec((HH, H), lambda i: (0, 0)),
        pl.BlockSpec((1, H), lambda i: (0, 0)),
    ],
    out_specs=pl.BlockSpec((NC, BN, HH), lambda i: (0, i, 0)),
    out_shape=jax.ShapeDtypeStruct((NC, NP, HH), jnp.float32),
)


def _layer1_body(s_ref, h_ref, s0_ref, wl_ref, wr_ref, b_ref, o_ref):
    sfull = jnp.concatenate([s_ref[0], s_ref[1]], axis=1)
    hfull = jnp.concatenate([h_ref[0], h_ref[1]], axis=1)
    deg = s0_ref[0][:, 2:3] + s0_ref[1][:, 2:3]
    inv = 1.0 / jnp.maximum(deg, 1.0)
    o = (jnp.dot(sfull * inv, wl_ref[...], preferred_element_type=jnp.float32)
         + jnp.dot(hfull, wr_ref[...], preferred_element_type=jnp.float32)
         + b_ref[...])
    o = jnp.maximum(o, 0.0)
    o_ref[0] = o[:, :HH]
    o_ref[1] = o[:, HH:]


_layer1 = pl.pallas_call(
    _layer1_body,
    grid=(NBN,),
    in_specs=[
        pl.BlockSpec((NC, BN, HH), lambda i: (0, i, 0)),
        pl.BlockSpec((NC, BN, HH), lambda i: (0, i, 0)),
        pl.BlockSpec((NC, BN, HH), lambda i: (0, i, 0)),
        pl.BlockSpec((H, H), lambda i: (0, 0)),
        pl.BlockSpec((H, H), lambda i: (0, 0)),
        pl.BlockSpec((1, H), lambda i: (0, 0)),
    ],
    out_specs=pl.BlockSpec((NC, BN, HH), lambda i: (0, i, 0)),
    out_shape=jax.ShapeDtypeStruct((NC, NP, HH), jnp.float32),
)


def _final_body(s_ref, h_ref, s0_ref, bt_ref, wl_ref, wr_ref, b_ref, o_ref,
                acc, cnt):
    i = pl.program_id(0)

    @pl.when(i == 0)
    def _():
        acc[...] = jnp.zeros_like(acc)
        cnt[...] = jnp.zeros_like(cnt)

    sfull = jnp.concatenate([s_ref[0], s_ref[1]], axis=1)
    hfull = jnp.concatenate([h_ref[0], h_ref[1]], axis=1)
    deg = s0_ref[0][:, 2:3] + s0_ref[1][:, 2:3]
    inv = 1.0 / jnp.maximum(deg, 1.0)
    h2 = (jnp.dot(sfull * inv, wl_ref[...], preferred_element_type=jnp.float32)
          + jnp.dot(hfull, wr_ref[...], preferred_element_type=jnp.float32)
          + b_ref[...])
    bt = bt_ref[0, 0, :]
    p = (lax.broadcasted_iota(jnp.int32, (G, BN), 0) == bt[None, :]).astype(
        jnp.float32)
    acc[...] += jnp.dot(p, h2, preferred_element_type=jnp.float32)
    cnt[...] += jnp.sum(p, axis=1, keepdims=True)

    @pl.when(i == NBN - 1)
    def _():
        o_ref[...] = acc[...] / jnp.maximum(cnt[...], 1.0)


_final = pl.pallas_call(
    _final_body,
    grid=(NBN,),
    in_specs=[
        pl.BlockSpec((NC, BN, HH), lambda i: (0, i, 0)),
        pl.BlockSpec((NC, BN, HH), lambda i: (0, i, 0)),
        pl.BlockSpec((NC, BN, HH), lambda i: (0, i, 0)),
        pl.BlockSpec((1, 1, BN), lambda i: (i, 0, 0)),
        pl.BlockSpec((H, H), lambda i: (0, 0)),
        pl.BlockSpec((H, H), lambda i: (0, 0)),
        pl.BlockSpec((1, H), lambda i: (0, 0)),
    ],
    out_specs=pl.BlockSpec((G, H), lambda i: (0, 0)),
    out_shape=jax.ShapeDtypeStruct((G, H), jnp.float32),
    scratch_shapes=[
        pltpu.VMEM((G, H), jnp.float32),
        pltpu.VMEM((G, 1), jnp.float32),
    ],
)


def kernel(x, edge_index, batch, W_l0, b0, W_r0, W_l1, b1, W_r1, W_l2, b2,
           W_r2):
    src = edge_index[0].reshape(NS, EPT2)
    dst = edge_index[1].reshape(NS, NBLK2, BLK2)
    src0 = edge_index[0].reshape(NC * NS, NBLK0, BLK0)
    dst0 = edge_index[1].reshape(NC * NS, NBLK0, BLK0)

    x0 = jnp.concatenate(
        [x, jnp.ones((N, 1), jnp.float32), jnp.zeros((N, HH - 3), jnp.float32)],
        axis=1)
    x0 = jnp.pad(x0, ((0, NP - N), (0, 0)))
    wl0 = jnp.pad(W_l0, ((0, HH - 2), (0, 0)))
    wr0 = jnp.pad(W_r0, ((0, HH - 2), (0, 0)))
    batch3 = jnp.pad(batch, (0, NP - N), constant_values=G).reshape(NBN, 1, BN)
    b0r, b1r, b2r = b0.reshape(1, H), b1.reshape(1, H), b2.reshape(1, H)

    s0 = _agg0(x0, src0, dst0)
    h0 = _layer0(s0, x0, wl0, wr0, b0r)
    s1 = _agg(h0, src, dst)
    h1 = _layer1(s1, h0, s0, W_l1, W_r1, b1r)
    s2 = _agg(h1, src, dst)
    return _final(s2, h1, s0, batch3, W_l2, W_r2, b2r)


# speedup vs baseline: 1.0266x; 1.0039x over previous
import functools

import jax
import jax.numpy as jnp
from jax import lax
from jax.experimental import pallas as pl
from jax.experimental.pallas import tpu as pltpu
from jax.experimental.pallas import tpu_sc as plsc

N = 10000
E = 160000
H = 256
HH = H // 2
G = 64

NP = 10240
NC, NS = 2, 16
STRIPE = NP // NS

EPT2 = E // NS
BLK2 = 80
NBLK2 = EPT2 // BLK2


def _agg_body(h_hbm, src_hbm, dst_hbm, out_hbm, src_v, dst_v, buf0,
              buf1, g0, g1, acc):
    c = lax.axis_index("c")
    s = lax.axis_index("s")

    pltpu.sync_copy(src_hbm.at[s], src_v)
    pltpu.sync_copy(dst_hbm.at[s], dst_v)

    z = jnp.zeros((16,), jnp.float32)

    def zero_row(r, _):
        for k in range(HH // 16):
            buf0[r, pl.ds(k * 16, 16)] = z
        return 0

    lax.fori_loop(0, BLK2, zero_row, 0)
    for q in range(STRIPE // BLK2):
        pltpu.sync_copy(buf0, acc.at[pl.ds(s * STRIPE + q * BLK2, BLK2)])
    rem = STRIPE % BLK2
    if rem:
        pltpu.sync_copy(
            buf0.at[pl.ds(0, rem)],
            acc.at[pl.ds(s * STRIPE + (STRIPE // BLK2) * BLK2, rem)])

    plsc.subcore_barrier()

    def gath(b, buf, sem):
        pltpu.async_copy(
            h_hbm.at[c].at[src_v.at[pl.ds(b * BLK2, BLK2)]], buf, sem)

    def wait_g(buf, sem):
        pltpu.make_async_copy(h_hbm.at[c], buf, sem).wait()

    def scat(b, buf):
        pltpu.sync_copy(buf, acc.at[dst_v.at[b]], add=True)

    gath(0, buf0, g0)

    def pair(t, _):
        e = 2 * t
        gath(e + 1, buf1, g1)
        wait_g(buf0, g0)
        scat(e, buf0)
        gath(e + 2, buf0, g0)
        wait_g(buf1, g1)
        scat(e + 1, buf1)
        return 0

    lax.fori_loop(0, (NBLK2 - 1) // 2, pair, 0)
    wait_g(buf0, g0)
    scat(NBLK2 - 1, buf0)

    plsc.subcore_barrier()

    pltpu.sync_copy(acc.at[pl.ds(s * STRIPE, STRIPE)],
                    out_hbm.at[c, pl.ds(s * STRIPE, STRIPE)])


@functools.lru_cache(maxsize=None)
def _get_agg():
    return pl.kernel(
        _agg_body,
        out_type=jax.ShapeDtypeStruct((NC, NP, HH), jnp.float32),
        mesh=plsc.VectorSubcoreMesh(core_axis_name="c", subcore_axis_name="s",
                                    num_cores=NC, num_subcores=NS),
        scratch_types=[
            pltpu.VMEM((EPT2,), jnp.int32),
            pltpu.VMEM((NBLK2, BLK2), jnp.int32),
            pltpu.VMEM((BLK2, HH), jnp.float32),
            pltpu.VMEM((BLK2, HH), jnp.float32),
            pltpu.SemaphoreType.DMA,
            pltpu.SemaphoreType.DMA,
            pltpu.VMEM_SHARED((NP, HH), jnp.float32),
        ],
    )


EPT0 = E // (NC * NS)
BLK0 = 125
NBLK0 = EPT0 // BLK0


def _agg0_body(x_hbm, src_hbm, dst_hbm, out_hbm, src_v, dst_v, buf0, buf1,
               g0, g1, acc):
    c = lax.axis_index("c")
    s = lax.axis_index("s")
    w = c * NS + s

    pltpu.sync_copy(src_hbm.at[w], src_v)
    pltpu.sync_copy(dst_hbm.at[w], dst_v)

    z = jnp.zeros((16,), jnp.float32)

    def zero_row(r, _):
        for k in range(HH // 16):
            buf0[r, pl.ds(k * 16, 16)] = z
        return 0

    lax.fori_loop(0, BLK0, zero_row, 0)
    for q in range(STRIPE // BLK0):
        pltpu.sync_copy(buf0, acc.at[pl.ds(s * STRIPE + q * BLK0, BLK0)])
    rem = STRIPE % BLK0
    if rem:
        pltpu.sync_copy(
            buf0.at[pl.ds(0, rem)],
            acc.at[pl.ds(s * STRIPE + (STRIPE // BLK0) * BLK0, rem)])

    plsc.subcore_barrier()

    def gath(b, buf, sem):
        return pltpu.async_copy(x_hbm.at[src_v.at[b]], buf, sem)

    def scat(b, buf):
        pltpu.sync_copy(buf, acc.at[dst_v.at[b]], add=True)

    gath(0, buf0, g0)

    def pair(t, _):
        e = 2 * t
        gath(e + 1, buf1, g1)
        pltpu.make_async_copy(x_hbm, buf0, g0).wait()
        scat(e, buf0)
        gath(e + 2, buf0, g0)
        pltpu.make_async_copy(x_hbm, buf1, g1).wait()
        scat(e + 1, buf1)
        return 0

    lax.fori_loop(0, (NBLK0 - 2) // 2, pair, 0)
    gath(NBLK0 - 1, buf1, g1)
    pltpu.make_async_copy(x_hbm, buf0, g0).wait()
    scat(NBLK0 - 2, buf0)
    pltpu.make_async_copy(x_hbm, buf1, g1).wait()
    scat(NBLK0 - 1, buf1)

    plsc.subcore_barrier()

    pltpu.sync_copy(acc.at[pl.ds(s * STRIPE, STRIPE)],
                    out_hbm.at[c, pl.ds(s * STRIPE, STRIPE)])


@functools.lru_cache(maxsize=None)
def _get_agg0():
    return pl.kernel(
        _agg0_body,
        out_type=jax.ShapeDtypeStruct((NC, NP, HH), jnp.float32),
        mesh=plsc.VectorSubcoreMesh(core_axis_name="c", subcore_axis_name="s",
                                    num_cores=NC, num_subcores=NS),
        scratch_types=[
            pltpu.VMEM((NBLK0, BLK0), jnp.int32),
            pltpu.VMEM((NBLK0, BLK0), jnp.int32),
            pltpu.VMEM((BLK0, HH), jnp.float32),
            pltpu.VMEM((BLK0, HH), jnp.float32),
            pltpu.SemaphoreType.DMA,
            pltpu.SemaphoreType.DMA,
            pltpu.VMEM_SHARED((NP, HH), jnp.float32),
        ],
    )


def _agg(h, src, dst):
    return _get_agg()(h, src, dst)


def _agg0(x0, src, dst):
    return _get_agg0()(x0, src, dst)


BN = 512
NBN = NP // BN


def _layer0_body(s_ref, x_ref, wl_ref, wr_ref, b_ref, o_ref):
    sfull = s_ref[0] + s_ref[1]
    deg = sfull[:, 2:3]
    inv = 1.0 / jnp.maximum(deg, 1.0)
    o = (jnp.dot(sfull * inv, wl_ref[...], preferred_element_type=jnp.float32)
         + jnp.dot(x_ref[...], wr_ref[...], preferred_element_type=jnp.float32)
         + b_ref[...])
    o = jnp.maximum(o, 0.0)
    o_ref[0] = o[:, :HH]
    o_ref[1] = o[:, HH:]


_layer0 = pl.pallas_call(
    _layer0_body,
    grid=(NBN,),
    in_specs=[
        pl.BlockSpec((NC, BN, HH), lambda i: (0, i, 0)),
        pl.BlockSpec((BN, HH), lambda i: (i, 0)),
        pl.BlockSpec((HH, H), lambda i: (0, 0)),
        pl.BlockSpec((HH, H), lambda i: (0, 0)),
        pl.BlockSpec((1, H), lambda i: (0, 0)),
    ],
    out_specs=pl.BlockSpec((NC, BN, HH), lambda i: (0, i, 0)),
    out_shape=jax.ShapeDtypeStruct((NC, NP, HH), jnp.float32),
)


def _layer1_body(s_ref, h_ref, s0_ref, wl_ref, wr_ref, b_ref, o_ref):
    sfull = jnp.concatenate([s_ref[0], s_ref[1]], axis=1)
    hfull = jnp.concatenate([h_ref[0], h_ref[1]], axis=1)
    deg = s0_ref[0][:, 2:3] + s0_ref[1][:, 2:3]
    inv = 1.0 / jnp.maximum(deg, 1.0)
    o = (jnp.dot(sfull * inv, wl_ref[...], preferred_element_type=jnp.float32)
         + jnp.dot(hfull, wr_ref[...], preferred_element_type=jnp.float32)
         + b_ref[...])
    o = jnp.maximum(o, 0.0)
    o_ref[0] = o[:, :HH]
    o_ref[1] = o[:, HH:]


_layer1 = pl.pallas_call(
    _layer1_body,
    grid=(NBN,),
    in_specs=[
        pl.BlockSpec((NC, BN, HH), lambda i: (0, i, 0)),
        pl.BlockSpec((NC, BN, HH), lambda i: (0, i, 0)),
        pl.BlockSpec((NC, BN, HH), lambda i: (0, i, 0)),
        pl.BlockSpec((H, H), lambda i: (0, 0)),
        pl.BlockSpec((H, H), lambda i: (0, 0)),
        pl.BlockSpec((1, H), lambda i: (0, 0)),
    ],
    out_specs=pl.BlockSpec((NC, BN, HH), lambda i: (0, i, 0)),
    out_shape=jax.ShapeDtypeStruct((NC, NP, HH), jnp.float32),
)


def _final_body(s_ref, h_ref, s0_ref, bt_ref, wl_ref, wr_ref, b_ref, o_ref,
                acc, cnt):
    i = pl.program_id(0)

    @pl.when(i == 0)
    def _():
        acc[...] = jnp.zeros_like(acc)
        cnt[...] = jnp.zeros_like(cnt)

    sfull = jnp.concatenate([s_ref[0], s_ref[1]], axis=1)
    hfull = jnp.concatenate([h_ref[0], h_ref[1]], axis=1)
    deg = s0_ref[0][:, 2:3] + s0_ref[1][:, 2:3]
    inv = 1.0 / jnp.maximum(deg, 1.0)
    h2 = (jnp.dot(sfull * inv, wl_ref[...], preferred_element_type=jnp.float32)
          + jnp.dot(hfull, wr_ref[...], preferred_element_type=jnp.float32)
          + b_ref[...])
    bt = bt_ref[0, 0, :]
    p = (lax.broadcasted_iota(jnp.int32, (G, BN), 0) == bt[None, :]).astype(
        jnp.float32)
    acc[...] += jnp.dot(p, h2, preferred_element_type=jnp.float32)
    cnt[...] += jnp.sum(p, axis=1, keepdims=True)

    @pl.when(i == NBN - 1)
    def _():
        o_ref[...] = acc[...] / jnp.maximum(cnt[...], 1.0)


_final = pl.pallas_call(
    _final_body,
    grid=(NBN,),
    in_specs=[
        pl.BlockSpec((NC, BN, HH), lambda i: (0, i, 0)),
        pl.BlockSpec((NC, BN, HH), lambda i: (0, i, 0)),
        pl.BlockSpec((NC, BN, HH), lambda i: (0, i, 0)),
        pl.BlockSpec((1, 1, BN), lambda i: (i, 0, 0)),
        pl.BlockSpec((H, H), lambda i: (0, 0)),
        pl.BlockSpec((H, H), lambda i: (0, 0)),
        pl.BlockSpec((1, H), lambda i: (0, 0)),
    ],
    out_specs=pl.BlockSpec((G, H), lambda i: (0, 0)),
    out_shape=jax.ShapeDtypeStruct((G, H), jnp.float32),
    scratch_shapes=[
        pltpu.VMEM((G, H), jnp.float32),
        pltpu.VMEM((G, 1), jnp.float32),
    ],
)


def kernel(x, edge_index, batch, W_l0, b0, W_r0, W_l1, b1, W_r1, W_l2, b2,
           W_r2):
    src = edge_index[0].reshape(NS, EPT2)
    dst = edge_index[1].reshape(NS, NBLK2, BLK2)
    src0 = edge_index[0].reshape(NC * NS, NBLK0, BLK0)
    dst0 = edge_index[1].reshape(NC * NS, NBLK0, BLK0)

    x0 = jnp.concatenate(
        [x, jnp.ones((N, 1), jnp.float32), jnp.zeros((N, HH - 3), jnp.float32)],
        axis=1)
    x0 = jnp.pad(x0, ((0, NP - N), (0, 0)))
    wl0 = jnp.pad(W_l0, ((0, HH - 2), (0, 0)))
    wr0 = jnp.pad(W_r0, ((0, HH - 2), (0, 0)))
    batch3 = jnp.pad(batch, (0, NP - N), constant_values=G).reshape(NBN, 1, BN)
    b0r, b1r, b2r = b0.reshape(1, H), b1.reshape(1, H), b2.reshape(1, H)

    s0 = _agg0(x0, src0, dst0)
    h0 = _layer0(s0, x0, wl0, wr0, b0r)
    s1 = _agg(h0, src, dst)
    h1 = _layer1(s1, h0, s0, W_l1, W_r1, b1r)
    s2 = _agg(h1, src, dst)
    return _final(s2, h1, s0, batch3, W_l2, W_r2, b2r)
